# Initial kernel scaffold; baseline (speedup 1.0000x reference)
import jax
import jax.numpy as jnp
from jax.experimental import pallas as pl


def kernel(z, edge_index, edge_attr, W1, b1, W2, b2, Wc, bc):
    raise NotImplementedError("write your pallas kernel here")


# speedup vs baseline: 27.6999x; 27.6999x over previous
import functools

import jax
import jax.numpy as jnp
from jax import lax
from jax.experimental import pallas as pl
from jax.experimental.pallas import tpu as pltpu
from jax.experimental.pallas import tpu_sc as plsc

NC = 2
NS = 16
L = 16
F = 4
G = NS // F
T = (NC * NS) // G
CH = 2000

_MESH = dict(core_axis_name="c", subcore_axis_name="s", num_cores=NC,
             num_subcores=NS)


def _zero_ref(ref, n):
    z = jnp.zeros((L,), jnp.float32)

    def body(i, _):
        ref[pl.ds(i * L, L)] = z
        return 0

    lax.fori_loop(0, n // L, body, 0)


def _deg_body(col_hbm, ea_hbm, degp_hbm, cbuf, ebuf, dacc):
    E = col_hbm.shape[0]
    n = degp_hbm.shape[1]
    c = lax.axis_index("c")
    s = lax.axis_index("s")
    wid = c * NS + s
    per = E // (NC * NS)
    base = wid * per
    _zero_ref(dacc, n)

    def piece(p, _):
        off = base + p * CH
        pltpu.sync_copy(col_hbm.at[pl.ds(off, CH)], cbuf)
        pltpu.sync_copy(ea_hbm.at[pl.ds(off, CH)], ebuf)

        def it(i, _):
            cc = cbuf[pl.ds(i * L, L)]
            ew = ebuf[pl.ds(i * L, L)]
            plsc.addupdate_scatter(dacc, [cc], ew)
            return 0

        lax.fori_loop(0, CH // L, it, 0)
        return 0

    lax.fori_loop(0, per // CH, piece, 0)
    pltpu.sync_copy(dacc, degp_hbm.at[wid])


def _deg_partials(col, edge_attr, n):
    E = col.shape[0]
    k = pl.kernel(
        _deg_body,
        out_type=jax.ShapeDtypeStruct((NC * NS, n), jnp.float32),
        mesh=plsc.VectorSubcoreMesh(**_MESH),
        compiler_params=pltpu.CompilerParams(needs_layout_passes=False),
        scratch_types=[
            pltpu.VMEM((CH,), jnp.int32),
            pltpu.VMEM((CH,), jnp.float32),
            pltpu.VMEM((n,), jnp.float32),
        ],
    )
    return k(col, edge_attr)


def _edge_loop(row_hbm, col_hbm, norm_src, xrs, ags, rbuf, cbuf, nbuf, c, s, E):
    g = s % G
    t_chunk = c * (T // NC) + s // G
    per = E // T
    base = t_chunk * per

    def piece(p, _):
        off = base + p * CH
        pltpu.sync_copy(row_hbm.at[pl.ds(off, CH)], rbuf)
        pltpu.sync_copy(col_hbm.at[pl.ds(off, CH)], cbuf)
        pltpu.sync_copy(norm_src.at[pl.ds(off, CH)], nbuf)

        def it(i, _):
            r = rbuf[pl.ds(i * L, L)]
            cc = cbuf[pl.ds(i * L, L)]
            nv = nbuf[pl.ds(i * L, L)]
            for j in range(F):
                xv = plsc.load_gather(xrs[j], [r])
                plsc.addupdate_scatter(ags[j], [cc], nv * xv)
            return 0

        lax.fori_loop(0, CH // L, it, 0)
        return 0

    lax.fori_loop(0, per // CH, piece, 0)
    return g, t_chunk


def _layer1_body(row_hbm, col_hbm, ea_hbm, dinv_hbm, xT_hbm, norm_hbm, agg_hbm,
                 dinv_v, xr0, xr1, xr2, xr3, ag0, ag1, ag2, ag3,
                 rbuf, cbuf, nbuf, wbuf):
    E = row_hbm.shape[0]
    n = dinv_hbm.shape[0]
    c = lax.axis_index("c")
    s = lax.axis_index("s")
    xrs = (xr0, xr1, xr2, xr3)
    ags = (ag0, ag1, ag2, ag3)
    g = s % G

    pltpu.sync_copy(dinv_hbm, dinv_v)
    for j in range(F):
        pltpu.sync_copy(xT_hbm.at[g * F + j], xrs[j])
        _zero_ref(ags[j], n)

    per_np = E // (2 * NS)
    base_np = c * (E // 2) + s * per_np

    def npiece(p, _):
        off = base_np + p * CH
        pltpu.sync_copy(row_hbm.at[pl.ds(off, CH)], rbuf)
        pltpu.sync_copy(col_hbm.at[pl.ds(off, CH)], cbuf)
        pltpu.sync_copy(ea_hbm.at[pl.ds(off, CH)], nbuf)

        def it(i, _):
            sl = pl.ds(i * L, L)
            r = rbuf[sl]
            cc = cbuf[sl]
            ew = nbuf[sl]
            dr = plsc.load_gather(dinv_v, [r])
            dc = plsc.load_gather(dinv_v, [cc])
            wbuf[sl] = dr * ew * dc
            return 0

        lax.fori_loop(0, CH // L, it, 0)
        pltpu.sync_copy(wbuf, norm_hbm.at[pl.ds(off, CH)])
        return 0

    lax.fori_loop(0, per_np // CH, npiece, 0)
    plsc.subcore_barrier()

    g, t_chunk = _edge_loop(row_hbm, col_hbm, norm_hbm, xrs, ags, rbuf, cbuf,
                            nbuf, c, s, E)
    for j in range(F):
        pltpu.sync_copy(ags[j], agg_hbm.at[t_chunk, g * F + j])


def _layer2_body(row_hbm, col_hbm, norm_hbm, xT_hbm, agg_hbm,
                 xr0, xr1, xr2, xr3, ag0, ag1, ag2, ag3, rbuf, cbuf, nbuf):
    E = row_hbm.shape[0]
    n = xT_hbm.shape[1]
    c = lax.axis_index("c")
    s = lax.axis_index("s")
    xrs = (xr0, xr1, xr2, xr3)
    ags = (ag0, ag1, ag2, ag3)
    g = s % G
    for j in range(F):
        pltpu.sync_copy(xT_hbm.at[g * F + j], xrs[j])
        _zero_ref(ags[j], n)
    g, t_chunk = _edge_loop(row_hbm, col_hbm, norm_hbm, xrs, ags, rbuf, cbuf,
                            nbuf, c, s, E)
    for j in range(F):
        pltpu.sync_copy(ags[j], agg_hbm.at[t_chunk, g * F + j])


def _sc_layer1(row, col, edge_attr, dinv, xT):
    E = row.shape[0]
    n = dinv.shape[0]
    vf = lambda shape: pltpu.VMEM(shape, jnp.float32)
    k = pl.kernel(
        _layer1_body,
        out_type=(jax.ShapeDtypeStruct((E,), jnp.float32),
                  jax.ShapeDtypeStruct((T, NS, n), jnp.float32)),
        mesh=plsc.VectorSubcoreMesh(**_MESH),
        compiler_params=pltpu.CompilerParams(needs_layout_passes=False),
        scratch_types=[vf((n,))] * 5 + [vf((n,))] * 4 +
                      [pltpu.VMEM((CH,), jnp.int32),
                       pltpu.VMEM((CH,), jnp.int32),
                       vf((CH,)), vf((CH,))],
    )
    return k(row, col, edge_attr, dinv, xT)


def _sc_layer2(row, col, norm, xT):
    E = row.shape[0]
    n = xT.shape[1]
    vf = lambda shape: pltpu.VMEM(shape, jnp.float32)
    k = pl.kernel(
        _layer2_body,
        out_type=jax.ShapeDtypeStruct((T, NS, n), jnp.float32),
        mesh=plsc.VectorSubcoreMesh(**_MESH),
        compiler_params=pltpu.CompilerParams(needs_layout_passes=False),
        scratch_types=[vf((n,))] * 8 +
                      [pltpu.VMEM((CH,), jnp.int32),
                       pltpu.VMEM((CH,), jnp.int32),
                       vf((CH,))],
    )
    return k(row, col, norm, xT)


def _tc1_body(z_ref, w1_ref, degp_ref, xt_ref, dinv_ref, sn_ref):
    deg = jnp.sum(degp_ref[...], axis=0, keepdims=True) + 1.0
    dinv = lax.rsqrt(deg)
    dinv_ref[...] = dinv
    sn_ref[...] = dinv * dinv
    xt_ref[...] = lax.dot_general(
        w1_ref[...], z_ref[...], (((0,), (1,)), ((), ())),
        preferred_element_type=jnp.float32)


def _tc2_body(agg_ref, xt_ref, sn_ref, b1_ref, w2_ref, out_ref):
    pre = (jnp.sum(agg_ref[...], axis=0) + sn_ref[...] * xt_ref[...]
           + b1_ref[...])
    x1t = jnp.maximum(pre, 0.0)
    out_ref[...] = lax.dot_general(
        w2_ref[...], x1t, (((0,), (0,)), ((), ())),
        preferred_element_type=jnp.float32)


def _tc3_body(agg_ref, xt_ref, sn_ref, b2_ref, wc_ref, bc_ref, out_ref):
    x2t = (jnp.sum(agg_ref[...], axis=0) + sn_ref[...] * xt_ref[...]
           + b2_ref[...])
    logits = lax.dot_general(
        wc_ref[...], x2t, (((0,), (0,)), ((), ())),
        preferred_element_type=jnp.float32) + bc_ref[...]
    m = jnp.max(logits, axis=0, keepdims=True)
    y = logits - m
    lse = jnp.log(jnp.sum(jnp.exp(y), axis=0, keepdims=True))
    out_ref[...] = jnp.transpose(y - lse)


def kernel(z, edge_index, edge_attr, W1, b1, W2, b2, Wc, bc):
    n, d = z.shape
    h = W1.shape[1]
    ncls = Wc.shape[1]

    row = edge_index[0]
    col = edge_index[1]
    degp = _deg_partials(col, edge_attr, n)

    xt1T, dinv2d, selfnorm = pl.pallas_call(
        _tc1_body,
        out_shape=(jax.ShapeDtypeStruct((h, n), jnp.float32),
                   jax.ShapeDtypeStruct((1, n), jnp.float32),
                   jax.ShapeDtypeStruct((1, n), jnp.float32)),
    )(z, W1, degp)

    norm, agg1 = _sc_layer1(row, col, edge_attr,
                            jnp.reshape(dinv2d, (n,)), xt1T)

    xt2T = pl.pallas_call(
        _tc2_body,
        out_shape=jax.ShapeDtypeStruct((h, n), jnp.float32),
    )(agg1, xt1T, selfnorm, jnp.reshape(b1, (h, 1)), W2)

    agg2 = _sc_layer2(row, col, norm, xt2T)

    out = pl.pallas_call(
        _tc3_body,
        out_shape=jax.ShapeDtypeStruct((n, ncls), jnp.float32),
    )(agg2, xt2T, selfnorm, jnp.reshape(b2, (h, 1)), Wc,
      jnp.reshape(bc, (ncls, 1)))
    return out


# speedup vs baseline: 68.9669x; 2.4898x over previous
import functools

import jax
import jax.numpy as jnp
from jax import lax
from jax.experimental import pallas as pl
from jax.experimental.pallas import tpu as pltpu
from jax.experimental.pallas import tpu_sc as plsc

NC = 2
NS = 16
L = 16
F = 4
G = NS // F
T = (NC * NS) // G
CH = 2000
U = 5

_MESH = dict(core_axis_name="c", subcore_axis_name="s", num_cores=NC,
             num_subcores=NS)
_PARAMS = dict(
    mesh=plsc.VectorSubcoreMesh(**_MESH),
    compiler_params=pltpu.CompilerParams(needs_layout_passes=False),
)


def _zero_refs(refs, n):
    z = jnp.zeros((L,), jnp.float32)

    @plsc.parallel_loop(0, n // L, 1, unroll=U)
    def _(i):
        for ref in refs:
            ref[pl.ds(i * L, L)] = z


def _ring(srcs_hbm, bufs2, sems2, base, npieces, body):
    def start(p):
        b = p % 2
        off = base + p * CH
        for src, dst in zip(srcs_hbm, bufs2[b]):
            pltpu.async_copy(src.at[pl.ds(off, CH)], dst, sems2[b])

    start(0)
    for p in range(npieces):
        b = p % 2
        if p + 1 < npieces:
            start(p + 1)
        off = base + p * CH
        for src, dst in zip(srcs_hbm, bufs2[b]):
            pltpu.make_async_copy(src.at[pl.ds(off, CH)], dst,
                                  sems2[b]).wait()
        body(p, *bufs2[b])


def _deg_body(col_hbm, ea_hbm, degp_hbm,
              cb0, eb0, cb1, eb1, dacc, sem0, sem1):
    E = col_hbm.shape[0]
    n = degp_hbm.shape[1]
    c = lax.axis_index("c")
    s = lax.axis_index("s")
    wid = c * NS + s
    per = E // (NC * NS)
    _zero_refs((dacc,), n)

    def piece(p, cb, eb):
        @plsc.parallel_loop(0, CH // L, 1, unroll=U)
        def _(i):
            sl = pl.ds(i * L, L)
            plsc.addupdate_scatter(dacc, [cb[sl]], eb[sl])

    _ring((col_hbm, ea_hbm), ((cb0, eb0), (cb1, eb1)), (sem0, sem1),
          wid * per, per // CH, piece)
    pltpu.sync_copy(dacc, degp_hbm.at[wid])


def _deg_partials(col, edge_attr, n):
    k = pl.kernel(
        _deg_body,
        out_type=jax.ShapeDtypeStruct((NC * NS, n), jnp.float32),
        scratch_types=[
            pltpu.VMEM((CH,), jnp.int32),
            pltpu.VMEM((CH,), jnp.float32),
            pltpu.VMEM((CH,), jnp.int32),
            pltpu.VMEM((CH,), jnp.float32),
            pltpu.VMEM((n,), jnp.float32),
            pltpu.SemaphoreType.DMA,
            pltpu.SemaphoreType.DMA,
        ],
        **_PARAMS,
    )
    return k(col, edge_attr)


def _edge_loop(row_hbm, col_hbm, norm_src, xrs, ags, bufs2, sems2, c, s, E):
    t_chunk = c * (T // NC) + s // G
    per = E // T

    def piece(p, rb, cb, nb):
        @plsc.parallel_loop(0, CH // L, 1, unroll=U)
        def _(i):
            sl = pl.ds(i * L, L)
            r = rb[sl]
            cc = cb[sl]
            nv = nb[sl]
            for j in range(F):
                xv = plsc.load_gather(xrs[j], [r])
                plsc.addupdate_scatter(ags[j], [cc], nv * xv)

    _ring((row_hbm, col_hbm, norm_src), bufs2, sems2,
          t_chunk * per, per // CH, piece)
    return t_chunk


def _layer1_body(row_hbm, col_hbm, ea_hbm, dinv_hbm, xT_hbm, norm_hbm, agg_hbm,
                 dinv_v, xr0, xr1, xr2, xr3, ag0, ag1, ag2, ag3,
                 rb0, cb0, nb0, rb1, cb1, nb1, wbuf, sem0, sem1, semx):
    E = row_hbm.shape[0]
    n = dinv_hbm.shape[0]
    c = lax.axis_index("c")
    s = lax.axis_index("s")
    xrs = (xr0, xr1, xr2, xr3)
    ags = (ag0, ag1, ag2, ag3)
    bufs2 = ((rb0, cb0, nb0), (rb1, cb1, nb1))
    sems2 = (sem0, sem1)
    g = s % G

    pltpu.async_copy(dinv_hbm, dinv_v, semx)
    for j in range(F):
        pltpu.async_copy(xT_hbm.at[g * F + j], xrs[j], semx)
    _zero_refs(ags, n)
    pltpu.make_async_copy(dinv_hbm, dinv_v, semx).wait()
    for j in range(F):
        pltpu.make_async_copy(xT_hbm.at[g * F + j], xrs[j], semx).wait()

    per_np = E // (2 * NS)
    base_np = c * (E // 2) + s * per_np

    def npiece(p, rb, cb, eb):
        @plsc.parallel_loop(0, CH // L, 1, unroll=U)
        def _(i):
            sl = pl.ds(i * L, L)
            dr = plsc.load_gather(dinv_v, [rb[sl]])
            dc = plsc.load_gather(dinv_v, [cb[sl]])
            wbuf[sl] = dr * eb[sl] * dc
        pltpu.sync_copy(wbuf, norm_hbm.at[pl.ds(base_np + p * CH, CH)])

    _ring((row_hbm, col_hbm, ea_hbm), bufs2, sems2, base_np, per_np // CH,
          npiece)
    plsc.subcore_barrier()

    t_chunk = _edge_loop(row_hbm, col_hbm, norm_hbm, xrs, ags, bufs2, sems2,
                         c, s, E)
    for j in range(F):
        pltpu.sync_copy(ags[j], agg_hbm.at[t_chunk, g * F + j])


def _layer2_body(row_hbm, col_hbm, norm_hbm, xT_hbm, agg_hbm,
                 xr0, xr1, xr2, xr3, ag0, ag1, ag2, ag3,
                 rb0, cb0, nb0, rb1, cb1, nb1, sem0, sem1, semx):
    E = row_hbm.shape[0]
    n = xT_hbm.shape[1]
    c = lax.axis_index("c")
    s = lax.axis_index("s")
    xrs = (xr0, xr1, xr2, xr3)
    ags = (ag0, ag1, ag2, ag3)
    bufs2 = ((rb0, cb0, nb0), (rb1, cb1, nb1))
    g = s % G
    for j in range(F):
        pltpu.async_copy(xT_hbm.at[g * F + j], xrs[j], semx)
    _zero_refs(ags, n)
    for j in range(F):
        pltpu.make_async_copy(xT_hbm.at[g * F + j], xrs[j], semx).wait()
    t_chunk = _edge_loop(row_hbm, col_hbm, norm_hbm, xrs, ags, bufs2,
                         (sem0, sem1), c, s, E)
    for j in range(F):
        pltpu.sync_copy(ags[j], agg_hbm.at[t_chunk, g * F + j])


def _sc_layer1(row, col, edge_attr, dinv, xT):
    E = row.shape[0]
    n = dinv.shape[0]
    vf = lambda shape: pltpu.VMEM(shape, jnp.float32)
    vi = lambda shape: pltpu.VMEM(shape, jnp.int32)
    k = pl.kernel(
        _layer1_body,
        out_type=(jax.ShapeDtypeStruct((E,), jnp.float32),
                  jax.ShapeDtypeStruct((T, NS, n), jnp.float32)),
        scratch_types=[vf((n,))] * 9 +
                      [vi((CH,)), vi((CH,)), vf((CH,)),
                       vi((CH,)), vi((CH,)), vf((CH,)), vf((CH,)),
                       pltpu.SemaphoreType.DMA, pltpu.SemaphoreType.DMA,
                       pltpu.SemaphoreType.DMA],
        **_PARAMS,
    )
    return k(row, col, edge_attr, dinv, xT)


def _sc_layer2(row, col, norm, xT):
    E = row.shape[0]
    n = xT.shape[1]
    vf = lambda shape: pltpu.VMEM(shape, jnp.float32)
    vi = lambda shape: pltpu.VMEM(shape, jnp.int32)
    k = pl.kernel(
        _layer2_body,
        out_type=jax.ShapeDtypeStruct((T, NS, n), jnp.float32),
        scratch_types=[vf((n,))] * 8 +
                      [vi((CH,)), vi((CH,)), vf((CH,)),
                       vi((CH,)), vi((CH,)), vf((CH,)),
                       pltpu.SemaphoreType.DMA, pltpu.SemaphoreType.DMA,
                       pltpu.SemaphoreType.DMA],
        **_PARAMS,
    )
    return k(row, col, norm, xT)


def _tc1_body(z_ref, w1_ref, degp_ref, xt_ref, dinv_ref, sn_ref):
    deg = jnp.sum(degp_ref[...], axis=0, keepdims=True) + 1.0
    dinv = lax.rsqrt(deg)
    dinv_ref[...] = dinv
    sn_ref[...] = dinv * dinv
    xt_ref[...] = lax.dot_general(
        w1_ref[...], z_ref[...], (((0,), (1,)), ((), ())),
        preferred_element_type=jnp.float32)


def _tc2_body(agg_ref, xt_ref, sn_ref, b1_ref, w2_ref, out_ref):
    pre = (jnp.sum(agg_ref[...], axis=0) + sn_ref[...] * xt_ref[...]
           + b1_ref[...])
    x1t = jnp.maximum(pre, 0.0)
    out_ref[...] = lax.dot_general(
        w2_ref[...], x1t, (((0,), (0,)), ((), ())),
        preferred_element_type=jnp.float32)


def _tc3_body(agg_ref, xt_ref, sn_ref, b2_ref, wc_ref, bc_ref, out_ref):
    x2t = (jnp.sum(agg_ref[...], axis=0) + sn_ref[...] * xt_ref[...]
           + b2_ref[...])
    logits = lax.dot_general(
        wc_ref[...], x2t, (((0,), (0,)), ((), ())),
        preferred_element_type=jnp.float32) + bc_ref[...]
    m = jnp.max(logits, axis=0, keepdims=True)
    y = logits - m
    lse = jnp.log(jnp.sum(jnp.exp(y), axis=0, keepdims=True))
    out_ref[...] = jnp.transpose(y - lse)


def kernel(z, edge_index, edge_attr, W1, b1, W2, b2, Wc, bc):
    n, d = z.shape
    h = W1.shape[1]
    ncls = Wc.shape[1]

    row = edge_index[0]
    col = edge_index[1]
    degp = _deg_partials(col, edge_attr, n)

    xt1T, dinv2d, selfnorm = pl.pallas_call(
        _tc1_body,
        out_shape=(jax.ShapeDtypeStruct((h, n), jnp.float32),
                   jax.ShapeDtypeStruct((1, n), jnp.float32),
                   jax.ShapeDtypeStruct((1, n), jnp.float32)),
    )(z, W1, degp)

    norm, agg1 = _sc_layer1(row, col, edge_attr,
                            jnp.reshape(dinv2d, (n,)), xt1T)

    xt2T = pl.pallas_call(
        _tc2_body,
        out_shape=jax.ShapeDtypeStruct((h, n), jnp.float32),
    )(agg1, xt1T, selfnorm, jnp.reshape(b1, (h, 1)), W2)

    agg2 = _sc_layer2(row, col, norm, xt2T)

    out = pl.pallas_call(
        _tc3_body,
        out_shape=jax.ShapeDtypeStruct((n, ncls), jnp.float32),
    )(agg2, xt2T, selfnorm, jnp.reshape(b2, (h, 1)), Wc,
      jnp.reshape(bc, (ncls, 1)))
    return out
